# Initial kernel scaffold; baseline (speedup 1.0000x reference)
import jax
import jax.numpy as jnp
from jax.experimental import pallas as pl


def kernel(x, wide_table, embed_table, W1, b1, W2, b2, W3, b3, fcW, fcb):
    raise NotImplementedError("write your pallas kernel here")


# speedup vs baseline: 7.0782x; 7.0782x over previous
import functools

import jax
import jax.numpy as jnp
import numpy as np
from jax import lax
from jax.experimental import pallas as pl
from jax.experimental.pallas import tpu as pltpu
from jax.experimental.pallas import tpu_sc as plsc

_F = 26
_EMBED = 16
_BATCH = 16384
_FIELD_DIM = 100000
_D_IN = _F * _EMBED

_NC, _NS = 2, 16
_NW = _NC * _NS
_TOTAL_ROWS = _BATCH * _F
_BPW = _TOTAL_ROWS // _NW
_CHUNK = 1664
_NCHUNK = _BPW // _CHUNK


def _gather_body(idx_hbm, table_hbm, out_hbm, idx_v, rows_v, sem):
    wid = lax.axis_index("s") * _NC + lax.axis_index("c")
    base = wid * _BPW
    pltpu.sync_copy(idx_hbm.at[pl.ds(base, _BPW)], idx_v)
    for ci in range(_NCHUNK):
        pltpu.async_copy(
            table_hbm.at[idx_v.at[pl.ds(ci * _CHUNK, _CHUNK)]], rows_v, sem
        ).wait()
        pltpu.sync_copy(rows_v, out_hbm.at[pl.ds(base + ci * _CHUNK, _CHUNK)])


@functools.cache
def _make_gather():
    return pl.kernel(
        _gather_body,
        out_type=jax.ShapeDtypeStruct((_TOTAL_ROWS, _EMBED), jnp.float32),
        mesh=plsc.VectorSubcoreMesh(core_axis_name="c", subcore_axis_name="s"),
        scratch_types=[
            pltpu.VMEM((_BPW,), jnp.int32),
            pltpu.VMEM((_CHUNK, _EMBED), jnp.float32),
            pltpu.SemaphoreType.DMA,
        ],
        compiler_params=pltpu.CompilerParams(use_tc_tiling_on_sc=False),
    )

_BLK = 2048


def _mlp_body(emb_ref, W1_ref, b1_ref, W2_ref, b2_ref, W3_ref, b3_ref,
              fca_ref, fcc_ref, fcb_ref, out_ref):
    emb = emb_ref[...]
    h = jnp.maximum(
        jnp.dot(emb, W1_ref[...], preferred_element_type=jnp.float32,
                precision=lax.Precision.HIGHEST) + b1_ref[...], 0.0)
    h = jnp.maximum(
        jnp.dot(h, W2_ref[...], preferred_element_type=jnp.float32,
                precision=lax.Precision.HIGHEST) + b2_ref[...], 0.0)
    h = jnp.maximum(
        jnp.dot(h, W3_ref[...], preferred_element_type=jnp.float32,
                precision=lax.Precision.HIGHEST) + b3_ref[...], 0.0)
    s = (jnp.sum(emb * fca_ref[...], axis=1, keepdims=True)
         + jnp.sum(h * fcc_ref[...], axis=1, keepdims=True)
         + fcb_ref[...])
    out_ref[...] = jax.nn.sigmoid(s)


def _mlp(emb, W1, b1, W2, b2, W3, b3, fca, fcc, fcb):
    grid = (_BATCH // _BLK,)
    full = lambda shape: pl.BlockSpec(shape, lambda i: (0, 0))
    return pl.pallas_call(
        _mlp_body,
        grid=grid,
        in_specs=[
            pl.BlockSpec((_BLK, _D_IN), lambda i: (i, 0)),
            full(W1.shape), full(b1.shape), full(W2.shape), full(b2.shape),
            full(W3.shape), full(b3.shape), full(fca.shape), full(fcc.shape),
            full(fcb.shape),
        ],
        out_specs=pl.BlockSpec((_BLK, 1), lambda i: (i, 0)),
        out_shape=jax.ShapeDtypeStruct((_BATCH, 1), jnp.float32),
    )(emb, W1, b1, W2, b2, W3, b3, fca, fcc, fcb)


_OFFSETS = np.arange(_F, dtype=np.int32) * _FIELD_DIM


def kernel(x, wide_table, embed_table, W1, b1, W2, b2, W3, b3, fcW, fcb):
    idx = (x + jnp.asarray(_OFFSETS)[None, :]).reshape(-1)
    emb_rows = _make_gather()(idx, embed_table)
    emb = emb_rows.reshape(_BATCH, _D_IN)
    out = _mlp(
        emb, W1, b1.reshape(1, -1), W2, b2.reshape(1, -1), W3,
        b3.reshape(1, -1), fcW[:_D_IN, 0].reshape(1, _D_IN),
        fcW[_D_IN:, 0].reshape(1, 32), fcb.reshape(1, 1),
    )
    return out
